# baseline (speedup 1.0000x reference)
import functools

import jax
import jax.numpy as jnp
from jax.experimental import pallas as pl
from jax.experimental.pallas import tpu as pltpu


def _mm_body(a_hbm, b_hbm, o_hbm, abuf, bbuf, obuf, asem, bsem, osem,
             *, nch, tot, ch, depth):
    g = pl.program_id(0)
    batch = g // nch
    chunk = g % nch
    row = chunk * ch

    def start_a(c):
        pltpu.make_async_copy(
            a_hbm.at[c // nch, pl.ds((c % nch) * ch, ch), :],
            abuf.at[c % depth],
            asem.at[c % depth],
        ).start()

    def start_b(k):
        pltpu.make_async_copy(b_hbm.at[k], bbuf.at[k % 2], bsem.at[k % 2]).start()

    @pl.when(g == 0)
    def _prologue():
        start_b(0)
        for c in range(min(depth - 1, tot)):
            start_a(c)

    @pl.when((chunk == 0) & (batch + 1 < o_hbm.shape[0]))
    def _next_b():
        start_b(batch + 1)

    @pl.when(chunk == 0)
    def _wait_b():
        pltpu.make_async_copy(b_hbm.at[batch], bbuf.at[batch % 2],
                              bsem.at[batch % 2]).wait()

    @pl.when(g >= 2)
    def _wait_out_slot():
        pltpu.make_async_copy(obuf.at[g % 2],
                              o_hbm.at[0, pl.ds(0, ch), :], osem.at[g % 2]).wait()

    pltpu.make_async_copy(a_hbm.at[batch, pl.ds(row, ch), :],
                          abuf.at[g % depth], asem.at[g % depth]).wait()

    obuf[g % 2] = jax.lax.dot_general(
        abuf[g % depth], bbuf[batch % 2],
        dimension_numbers=(((1,), (0,)), ((), ())),
        preferred_element_type=jnp.float32,
        precision=jax.lax.Precision.DEFAULT,
    )

    pltpu.make_async_copy(obuf.at[g % 2], o_hbm.at[batch, pl.ds(row, ch), :],
                          osem.at[g % 2]).start()

    @pl.when(g + depth - 1 < tot)
    def _next_a():
        start_a(g + depth - 1)

    @pl.when(g == tot - 1)
    def _epilogue():
        pltpu.make_async_copy(obuf.at[(g - 1) % 2],
                              o_hbm.at[0, pl.ds(0, ch), :], osem.at[(g - 1) % 2]).wait()
        pltpu.make_async_copy(obuf.at[g % 2],
                              o_hbm.at[0, pl.ds(0, ch), :], osem.at[g % 2]).wait()


@functools.partial(jax.jit, static_argnames=("ch", "depth"))
def _batched_mm(a3, b3, ch=512, depth=4):
    nb, m, k = a3.shape
    n = b3.shape[-1]
    nch = m // ch
    tot = nb * nch
    body = functools.partial(_mm_body, nch=nch, tot=tot, ch=ch, depth=depth)
    return pl.pallas_call(
        body,
        grid=(tot,),
        in_specs=[
            pl.BlockSpec(memory_space=pltpu.HBM),
            pl.BlockSpec(memory_space=pltpu.HBM),
        ],
        out_specs=pl.BlockSpec(memory_space=pltpu.HBM),
        out_shape=jax.ShapeDtypeStruct((nb, m, n), jnp.float32),
        scratch_shapes=[
            pltpu.VMEM((depth, ch, k), jnp.float32),
            pltpu.VMEM((2, k, n), jnp.float32),
            pltpu.VMEM((2, ch, n), jnp.float32),
            pltpu.SemaphoreType.DMA((depth,)),
            pltpu.SemaphoreType.DMA((2,)),
            pltpu.SemaphoreType.DMA((2,)),
        ],
        compiler_params=pltpu.CompilerParams(
            dimension_semantics=("arbitrary",),
        ),
    )(a3, b3)


def kernel(a, b):
    B1, B2, M, K = a.shape
    N = b.shape[-1]
    a3 = a.reshape(B1 * B2, M, K)
    b3 = b.reshape(B1 * B2, K, N)
    out = _batched_mm(a3, b3, ch=min(512, M), depth=6)
    return out.reshape(B1, B2, M, N)


# speedup vs baseline: 1.0083x; 1.0083x over previous
import functools

import jax
import jax.numpy as jnp
from jax.experimental import pallas as pl
from jax.experimental.pallas import tpu as pltpu


def _mm_body(a_hbm, b_hbm, o_hbm, abuf, bbuf, obuf, asem, bsem, osem,
             *, nch, tot, ch, depth):
    g = pl.program_id(0)
    batch = g // nch
    chunk = g % nch
    row = chunk * ch

    def start_a(c):
        pltpu.make_async_copy(
            a_hbm.at[c // nch, pl.ds((c % nch) * ch, ch), :],
            abuf.at[c % depth],
            asem.at[c % depth],
        ).start()

    def start_b(k):
        pltpu.make_async_copy(b_hbm.at[k], bbuf.at[k % 2], bsem.at[k % 2]).start()

    @pl.when(g == 0)
    def _prologue():
        start_b(0)
        for c in range(min(depth - 1, tot)):
            start_a(c)

    @pl.when((chunk == 0) & (batch + 1 < o_hbm.shape[0]))
    def _next_b():
        start_b(batch + 1)

    @pl.when(chunk == 0)
    def _wait_b():
        pltpu.make_async_copy(b_hbm.at[batch], bbuf.at[batch % 2],
                              bsem.at[batch % 2]).wait()

    @pl.when(g >= 2)
    def _wait_out_slot():
        pltpu.make_async_copy(obuf.at[g % 2],
                              o_hbm.at[0, pl.ds(0, ch), :], osem.at[g % 2]).wait()

    pltpu.make_async_copy(a_hbm.at[batch, pl.ds(row, ch), :],
                          abuf.at[g % depth], asem.at[g % depth]).wait()

    obuf[g % 2] = jax.lax.dot_general(
        abuf[g % depth], bbuf[batch % 2],
        dimension_numbers=(((1,), (0,)), ((), ())),
        preferred_element_type=jnp.float32,
        precision=jax.lax.Precision.DEFAULT,
    )

    pltpu.make_async_copy(obuf.at[g % 2], o_hbm.at[batch, pl.ds(row, ch), :],
                          osem.at[g % 2]).start()

    @pl.when(g + depth - 1 < tot)
    def _next_a():
        start_a(g + depth - 1)

    @pl.when(g == tot - 1)
    def _epilogue():
        pltpu.make_async_copy(obuf.at[(g - 1) % 2],
                              o_hbm.at[0, pl.ds(0, ch), :], osem.at[(g - 1) % 2]).wait()
        pltpu.make_async_copy(obuf.at[g % 2],
                              o_hbm.at[0, pl.ds(0, ch), :], osem.at[g % 2]).wait()


@functools.partial(jax.jit, static_argnames=("ch", "depth"))
def _batched_mm(a3, b3, ch=512, depth=4):
    nb, m, k = a3.shape
    n = b3.shape[-1]
    nch = m // ch
    tot = nb * nch
    body = functools.partial(_mm_body, nch=nch, tot=tot, ch=ch, depth=depth)
    return pl.pallas_call(
        body,
        grid=(tot,),
        in_specs=[
            pl.BlockSpec(memory_space=pltpu.HBM),
            pl.BlockSpec(memory_space=pltpu.HBM),
        ],
        out_specs=pl.BlockSpec(memory_space=pltpu.HBM),
        out_shape=jax.ShapeDtypeStruct((nb, m, n), jnp.float32),
        scratch_shapes=[
            pltpu.VMEM((depth, ch, k), jnp.float32),
            pltpu.VMEM((2, k, n), jnp.float32),
            pltpu.VMEM((2, ch, n), jnp.float32),
            pltpu.SemaphoreType.DMA((depth,)),
            pltpu.SemaphoreType.DMA((2,)),
            pltpu.SemaphoreType.DMA((2,)),
        ],
        compiler_params=pltpu.CompilerParams(
            dimension_semantics=("arbitrary",),
        ),
    )(a3, b3)


def kernel(a, b):
    B1, B2, M, K = a.shape
    N = b.shape[-1]
    a3 = a.reshape(B1 * B2, M, K)
    b3 = b.reshape(B1 * B2, K, N)
    out = _batched_mm(a3, b3, ch=min(256, M), depth=8)
    return out.reshape(B1, B2, M, N)
